# baseline (speedup 1.0000x reference)
import functools

import jax
import jax.numpy as jnp
from jax import lax
from jax.experimental import pallas as pl
from jax.experimental.pallas import tpu as pltpu
from jax.experimental.pallas import tpu_sc as plsc

NUM_WORKERS = 32
CHUNK = 128
EMB_D = 64


def _emb_body(n_chunks, ids_hbm, table_hbm, out_hbm, idx_v, rows_v, gsem):
    c = lax.axis_index("c")
    s = lax.axis_index("s")
    wid = s * 2 + c
    per_w = n_chunks * CHUNK
    base = wid * per_w
    pltpu.sync_copy(ids_hbm.at[wid], idx_v)

    def step(j, carry):
        pltpu.async_copy(table_hbm.at[idx_v.at[j]], rows_v, gsem).wait()
        pltpu.sync_copy(rows_v, out_hbm.at[pl.ds(base + j * CHUNK, CHUNK)])
        return carry

    lax.fori_loop(0, n_chunks, step, 0)


def kernel(token_ids, weight):
    B, S = token_ids.shape
    n_tok = B * S
    assert n_tok % (NUM_WORKERS * CHUNK) == 0
    n_chunks = n_tok // (NUM_WORKERS * CHUNK)
    ids = token_ids.astype(jnp.int32).reshape(NUM_WORKERS, n_chunks, CHUNK)

    mesh = plsc.VectorSubcoreMesh(core_axis_name="c", subcore_axis_name="s")
    run = pl.kernel(
        functools.partial(_emb_body, n_chunks),
        out_type=jax.ShapeDtypeStruct((n_tok, EMB_D), jnp.float32),
        mesh=mesh,
        scratch_types=[
            pltpu.VMEM((n_chunks, CHUNK), jnp.int32),
            pltpu.VMEM((CHUNK, EMB_D), jnp.float32),
            pltpu.SemaphoreType.DMA,
        ],
        compiler_params=pltpu.CompilerParams(use_tc_tiling_on_sc=False),
    )
    out = run(ids, weight)
    return out.reshape(B, S, EMB_D)


# speedup vs baseline: 1.1080x; 1.1080x over previous
import functools
import jax, jax.numpy as jnp
from jax import lax
from jax.experimental import pallas as pl
from jax.experimental.pallas import tpu as pltpu
from jax.experimental.pallas import tpu_sc as plsc

NUM_WORKERS = 32
CHUNK = 128
EMB_D = 64
NB = 4


def _body(n_chunks, ids_hbm, table_hbm, out_hbm,
          idx_v, rows0, rows1, gsem0, gsem1, ssem0, ssem1):
    c = lax.axis_index("c")
    s = lax.axis_index("s")
    wid = s * 2 + c
    base = wid * n_chunks * CHUNK
    ngroups = n_chunks // NB
    npairs = ngroups // 2

    pltpu.sync_copy(ids_hbm.at[wid], idx_v)

    def gather(j, rows, b, gsem):
        return pltpu.async_copy(table_hbm.at[idx_v.at[j]], rows.at[b], gsem.at[b])

    def wait_gather(rows, b, gsem):
        pltpu.make_async_copy(table_hbm.at[idx_v.at[0]], rows.at[b], gsem.at[b]).wait()

    def store(j, rows, b, ssem):
        return pltpu.async_copy(
            rows.at[b], out_hbm.at[pl.ds(base + j * CHUNK, CHUNK)], ssem.at[b])

    def wait_store(rows, b, ssem):
        pltpu.make_async_copy(
            rows.at[b], out_hbm.at[pl.ds(base, CHUNK)], ssem.at[b]).wait()

    for b in range(NB):
        gather(b, rows0, b, gsem0)

    def pair(i, carry):
        g0 = 2 * i
        g1 = g0 + 1
        for b in range(NB):
            @pl.when(i > 0)
            def _():
                wait_store(rows1, b, ssem1)
            gather(g1 * NB + b, rows1, b, gsem1)
        for b in range(NB):
            wait_gather(rows0, b, gsem0)
            store(g0 * NB + b, rows0, b, ssem0)
        for b in range(NB):
            @pl.when(g0 + 2 < ngroups)
            def _():
                wait_store(rows0, b, ssem0)
                gather((g0 + 2) * NB + b, rows0, b, gsem0)
        for b in range(NB):
            wait_gather(rows1, b, gsem1)
            store(g1 * NB + b, rows1, b, ssem1)
        return carry

    lax.fori_loop(0, npairs, pair, 0)
    for b in range(NB):
        wait_store(rows0, b, ssem0)
        wait_store(rows1, b, ssem1)


def kernel(token_ids, weight):
    B, S = token_ids.shape
    n_tok = B * S
    n_chunks = n_tok // (NUM_WORKERS * CHUNK)
    assert n_chunks % (2 * NB) == 0
    ids = token_ids.astype(jnp.int32).reshape(NUM_WORKERS, n_chunks, CHUNK)
    mesh = plsc.VectorSubcoreMesh(core_axis_name="c", subcore_axis_name="s")
    run = pl.kernel(
        functools.partial(_body, n_chunks),
        out_type=jax.ShapeDtypeStruct((n_tok, EMB_D), jnp.float32),
        mesh=mesh,
        scratch_types=[
            pltpu.VMEM((n_chunks, CHUNK), jnp.int32),
            pltpu.VMEM((NB, CHUNK, EMB_D), jnp.float32),
            pltpu.VMEM((NB, CHUNK, EMB_D), jnp.float32),
            pltpu.SemaphoreType.DMA((NB,)),
            pltpu.SemaphoreType.DMA((NB,)),
            pltpu.SemaphoreType.DMA((NB,)),
            pltpu.SemaphoreType.DMA((NB,)),
        ],
        compiler_params=pltpu.CompilerParams(use_tc_tiling_on_sc=False),
    )
    return run(ids, weight).reshape(B, S, EMB_D)


# speedup vs baseline: 1.1143x; 1.0057x over previous
import functools
import jax, jax.numpy as jnp
from jax import lax
from jax.experimental import pallas as pl
from jax.experimental.pallas import tpu as pltpu
from jax.experimental.pallas import tpu_sc as plsc

NUM_WORKERS = 32
CHUNK = 512
EMB_D = 64
NB = 1


def _body(n_chunks, ids_hbm, table_hbm, out_hbm,
          idx_v, rows0, rows1, gsem0, gsem1, ssem0, ssem1):
    c = lax.axis_index("c")
    s = lax.axis_index("s")
    wid = s * 2 + c
    base = wid * n_chunks * CHUNK
    ngroups = n_chunks // NB
    npairs = ngroups // 2

    pltpu.sync_copy(ids_hbm.at[wid], idx_v)

    def gather(j, rows, b, gsem):
        return pltpu.async_copy(table_hbm.at[idx_v.at[j]], rows.at[b], gsem.at[b])

    def wait_gather(rows, b, gsem):
        pltpu.make_async_copy(table_hbm.at[idx_v.at[0]], rows.at[b], gsem.at[b]).wait()

    def store(j, rows, b, ssem):
        return pltpu.async_copy(
            rows.at[b], out_hbm.at[pl.ds(base + j * CHUNK, CHUNK)], ssem.at[b])

    def wait_store(rows, b, ssem):
        pltpu.make_async_copy(
            rows.at[b], out_hbm.at[pl.ds(base, CHUNK)], ssem.at[b]).wait()

    for b in range(NB):
        gather(b, rows0, b, gsem0)

    def pair(i, carry):
        g0 = 2 * i
        g1 = g0 + 1
        for b in range(NB):
            @pl.when(i > 0)
            def _():
                wait_store(rows1, b, ssem1)
            gather(g1 * NB + b, rows1, b, gsem1)
        for b in range(NB):
            wait_gather(rows0, b, gsem0)
            store(g0 * NB + b, rows0, b, ssem0)
        for b in range(NB):
            @pl.when(g0 + 2 < ngroups)
            def _():
                wait_store(rows0, b, ssem0)
                gather((g0 + 2) * NB + b, rows0, b, gsem0)
        for b in range(NB):
            wait_gather(rows1, b, gsem1)
            store(g1 * NB + b, rows1, b, ssem1)
        return carry

    lax.fori_loop(0, npairs, pair, 0)
    for b in range(NB):
        wait_store(rows0, b, ssem0)
        wait_store(rows1, b, ssem1)


def kernel(token_ids, weight):
    B, S = token_ids.shape
    n_tok = B * S
    n_chunks = n_tok // (NUM_WORKERS * CHUNK)
    assert n_chunks % (2 * NB) == 0
    ids = token_ids.astype(jnp.int32).reshape(NUM_WORKERS, n_chunks, CHUNK)
    mesh = plsc.VectorSubcoreMesh(core_axis_name="c", subcore_axis_name="s")
    run = pl.kernel(
        functools.partial(_body, n_chunks),
        out_type=jax.ShapeDtypeStruct((n_tok, EMB_D), jnp.float32),
        mesh=mesh,
        scratch_types=[
            pltpu.VMEM((n_chunks, CHUNK), jnp.int32),
            pltpu.VMEM((NB, CHUNK, EMB_D), jnp.float32),
            pltpu.VMEM((NB, CHUNK, EMB_D), jnp.float32),
            pltpu.SemaphoreType.DMA((NB,)),
            pltpu.SemaphoreType.DMA((NB,)),
            pltpu.SemaphoreType.DMA((NB,)),
            pltpu.SemaphoreType.DMA((NB,)),
        ],
        compiler_params=pltpu.CompilerParams(use_tc_tiling_on_sc=False),
    )
    return run(ids, weight).reshape(B, S, EMB_D)
